# baseline (speedup 1.0000x reference)
import functools

import jax
import jax.numpy as jnp
from jax import lax
from jax.experimental import pallas as pl
from jax.experimental.pallas import tpu as pltpu
from jax.experimental.pallas import tpu_sc as plsc

N = 10000
D_IN = 128
D_H = 128
D_OUT = 64

NC = 2
NS = 16
NW = NC * NS
K = 128
CPW = 80
E_PAD = NW * CPW * K
N_PAD = 10240
RPS = N_PAD // NS
CW = 16

_mesh = plsc.VectorSubcoreMesh(core_axis_name="c", subcore_axis_name="s")


def _hist_body(dst_hbm, zeros_hbm, ones_hbm, out_hbm, dst_v, ones_v, cnt_sh):
    c = lax.axis_index("c")
    s = lax.axis_index("s")
    w = c * NS + s
    sl = pl.ds(s * RPS, RPS)
    pltpu.sync_copy(zeros_hbm.at[sl], cnt_sh.at[sl])
    pltpu.sync_copy(ones_hbm, ones_v)
    pltpu.sync_copy(dst_hbm.at[w], dst_v)
    plsc.subcore_barrier()

    def body(j, carry):
        pltpu.sync_copy(ones_v, cnt_sh.at[dst_v.at[j]], add=True)
        return carry

    lax.fori_loop(0, CPW, body, 0)
    plsc.subcore_barrier()
    pltpu.sync_copy(cnt_sh.at[sl], out_hbm.at[c].at[sl])


_hist = pl.kernel(
    _hist_body,
    out_type=jax.ShapeDtypeStruct((NC, N_PAD, CW), jnp.float32),
    mesh=_mesh,
    compiler_params=pltpu.CompilerParams(use_tc_tiling_on_sc=False),
    scratch_types=[
        pltpu.VMEM((CPW, K), jnp.int32),
        pltpu.VMEM((K, CW), jnp.float32),
        pltpu.VMEM_SHARED((N_PAD, CW), jnp.float32),
    ],
)


IDX_SHIFT = 14
IDX_MASK = (1 << IDX_SHIFT) - 1


def _agg_body(h_hbm, pidx_hbm, zeros_hbm, out_hbm,
              idx_v, sstg, dstg, rows0, rows1,
              g0, g1, acc_sh):
    rows = (rows0, rows1)
    gsem = (g0, g1)
    c = lax.axis_index("c")
    s = lax.axis_index("s")
    w = c * NS + s
    sl = pl.ds(s * RPS, RPS)

    pltpu.sync_copy(zeros_hbm.at[sl], acc_sh.at[sl])

    pltpu.sync_copy(pidx_hbm.at[w], idx_v)
    plsc.subcore_barrier()

    def decode(j, p):
        for i in range(K // 16):
            ds = pl.ds(i * 16, 16)
            pk = idx_v[j, ds]
            sstg[p, ds] = jnp.right_shift(pk, IDX_SHIFT)
            dstg[p, ds] = jnp.bitwise_and(pk, IDX_MASK)

    def gather_start(b):
        pltpu.async_copy(h_hbm.at[sstg.at[b]], rows[b], gsem[b])

    def gather_wait(b):
        pltpu.make_async_copy(h_hbm.at[sstg.at[b]], rows[b], gsem[b]).wait()

    def scatter(b):
        pltpu.sync_copy(rows[b], acc_sh.at[dstg.at[b]], add=True)

    decode(0, 0)
    gather_start(0)

    def pair_body(g, carry):
        for b in range(2):
            j = 2 * g + b
            gather_wait(b)
            decode(j + 1, 1 - b)
            gather_start(1 - b)
            scatter(b)
        return carry

    lax.fori_loop(0, CPW // 2 - 1, pair_body, 0)
    gather_wait(0)
    decode(CPW - 1, 1)
    gather_start(1)
    scatter(0)
    gather_wait(1)
    scatter(1)
    plsc.subcore_barrier()
    pltpu.sync_copy(acc_sh.at[sl], out_hbm.at[c].at[sl])


def _make_agg(d):
    return pl.kernel(
        _agg_body,
        out_type=jax.ShapeDtypeStruct((NC, N_PAD, d), jnp.float32),
        mesh=_mesh,
        compiler_params=pltpu.CompilerParams(use_tc_tiling_on_sc=False),
        scratch_types=[
            pltpu.VMEM((CPW, K), jnp.int32),
            pltpu.VMEM((2, K), jnp.int32),
            pltpu.VMEM((2, K), jnp.int32),
            pltpu.VMEM((K, d), jnp.float32),
            pltpu.VMEM((K, d), jnp.float32),
            pltpu.SemaphoreType.DMA,
            pltpu.SemaphoreType.DMA,
            pltpu.VMEM_SHARED((N_PAD, d), jnp.float32),
        ],
    )


_agg128 = _make_agg(D_H)
_agg64 = _make_agg(D_OUT)


BLK = 512
GRID = N_PAD // BLK


def _deg_mm_body(c0_ref, c1_ref, x_ref, w_ref, h_ref, dinv_ref):
    cnt = c0_ref[:, 0:1] + c1_ref[:, 0:1] + 1.0
    d = lax.rsqrt(cnt)
    h = jnp.dot(x_ref[...], w_ref[...], preferred_element_type=jnp.float32)
    h_ref[...] = h * d
    dinv_ref[...] = jnp.broadcast_to(d, (BLK, CW))


_deg_mm = pl.pallas_call(
    _deg_mm_body,
    grid=(GRID,),
    in_specs=[
        pl.BlockSpec((BLK, CW), lambda i: (i, 0)),
        pl.BlockSpec((BLK, CW), lambda i: (i, 0)),
        pl.BlockSpec((BLK, D_IN), lambda i: (i, 0)),
        pl.BlockSpec((D_IN, D_H), lambda i: (0, 0)),
    ],
    out_specs=[
        pl.BlockSpec((BLK, D_H), lambda i: (i, 0)),
        pl.BlockSpec((BLK, CW), lambda i: (i, 0)),
    ],
    out_shape=[
        jax.ShapeDtypeStruct((N_PAD, D_H), jnp.float32),
        jax.ShapeDtypeStruct((N_PAD, CW), jnp.float32),
    ],
)


def _mid_body(p0_ref, p1_ref, hp_ref, dinv_ref, b_ref, w_ref, out_ref):
    d = dinv_ref[:, 0:1]
    r = jnp.maximum((p0_ref[...] + p1_ref[...] + hp_ref[...]) * d + b_ref[...], 0.0)
    out_ref[...] = jnp.dot(r, w_ref[...], preferred_element_type=jnp.float32) * d


_mid = pl.pallas_call(
    _mid_body,
    grid=(GRID,),
    in_specs=[
        pl.BlockSpec((BLK, D_H), lambda i: (i, 0)),
        pl.BlockSpec((BLK, D_H), lambda i: (i, 0)),
        pl.BlockSpec((BLK, D_H), lambda i: (i, 0)),
        pl.BlockSpec((BLK, CW), lambda i: (i, 0)),
        pl.BlockSpec((1, D_H), lambda i: (0, 0)),
        pl.BlockSpec((D_H, D_OUT), lambda i: (0, 0)),
    ],
    out_specs=pl.BlockSpec((BLK, D_OUT), lambda i: (i, 0)),
    out_shape=jax.ShapeDtypeStruct((N_PAD, D_OUT), jnp.float32),
)


def _final_body(q0_ref, q1_ref, hp_ref, dinv_ref, b_ref, out_ref):
    d = dinv_ref[:, 0:1]
    out_ref[...] = (q0_ref[...] + q1_ref[...] + hp_ref[...]) * d + b_ref[...]


_final = pl.pallas_call(
    _final_body,
    grid=(GRID,),
    in_specs=[
        pl.BlockSpec((BLK, D_OUT), lambda i: (i, 0)),
        pl.BlockSpec((BLK, D_OUT), lambda i: (i, 0)),
        pl.BlockSpec((BLK, D_OUT), lambda i: (i, 0)),
        pl.BlockSpec((BLK, CW), lambda i: (i, 0)),
        pl.BlockSpec((1, D_OUT), lambda i: (0, 0)),
    ],
    out_specs=pl.BlockSpec((BLK, D_OUT), lambda i: (i, 0)),
    out_shape=jax.ShapeDtypeStruct((N_PAD, D_OUT), jnp.float32),
)


def kernel(x, edge_index, train_mask, labels, W1, b1, W2, b2):
    del train_mask, labels
    padv = jnp.full((E_PAD - edge_index.shape[1],), N_PAD - 1, dtype=jnp.int32)
    src = jnp.concatenate([edge_index[0], padv]).reshape(NW, CPW, K)
    dst = jnp.concatenate([edge_index[1], padv]).reshape(NW, CPW, K)
    pidx = jnp.bitwise_or(jnp.left_shift(src, IDX_SHIFT), dst)

    x_pad = jnp.pad(x, ((0, N_PAD - N), (0, 0)))
    zeros_cnt = jnp.zeros((N_PAD, CW), jnp.float32)
    ones_k = jnp.ones((K, CW), jnp.float32)
    zeros_h = jnp.zeros((N_PAD, D_H), jnp.float32)
    zeros_o = jnp.zeros((N_PAD, D_OUT), jnp.float32)

    counts = _hist(dst, zeros_cnt, ones_k)
    h1p, dinv = _deg_mm(counts[0], counts[1], x_pad, W1)
    p = _agg128(h1p, pidx, zeros_h)
    h2p = _mid(p[0], p[1], h1p, dinv, b1.reshape(1, D_H), W2)
    q = _agg64(h2p, pidx, zeros_o)
    out = _final(q[0], q[1], h2p, dinv, b2.reshape(1, D_OUT))
    return out[:N]


# speedup vs baseline: 1.1360x; 1.1360x over previous
import functools

import jax
import jax.numpy as jnp
from jax import lax
from jax.experimental import pallas as pl
from jax.experimental.pallas import tpu as pltpu
from jax.experimental.pallas import tpu_sc as plsc

N = 10000
D_IN = 128
D_H = 128
D_OUT = 64

NC = 2
NS = 16
NW = NC * NS
K = 128
CPW = 80
E_PAD = NW * CPW * K
N_PAD = 10240
RPS = N_PAD // NS
CW = 16

_mesh = plsc.VectorSubcoreMesh(core_axis_name="c", subcore_axis_name="s")


def _hist_body(dst_hbm, zeros_hbm, ones_hbm, out_hbm, dst_v, ones_v, cnt_sh):
    c = lax.axis_index("c")
    s = lax.axis_index("s")
    w = c * NS + s
    sl = pl.ds(s * RPS, RPS)
    pltpu.sync_copy(zeros_hbm.at[sl], cnt_sh.at[sl])
    pltpu.sync_copy(ones_hbm, ones_v)
    pltpu.sync_copy(dst_hbm.at[w], dst_v)
    plsc.subcore_barrier()

    def body(j, carry):
        pltpu.sync_copy(ones_v, cnt_sh.at[dst_v.at[j]], add=True)
        return carry

    lax.fori_loop(0, CPW, body, 0)
    plsc.subcore_barrier()
    pltpu.sync_copy(cnt_sh.at[sl], out_hbm.at[c].at[sl])


_hist = pl.kernel(
    _hist_body,
    out_type=jax.ShapeDtypeStruct((NC, N_PAD, CW), jnp.float32),
    mesh=_mesh,
    compiler_params=pltpu.CompilerParams(use_tc_tiling_on_sc=False),
    scratch_types=[
        pltpu.VMEM((CPW, K), jnp.int32),
        pltpu.VMEM((K, CW), jnp.float32),
        pltpu.VMEM_SHARED((N_PAD, CW), jnp.float32),
    ],
)


IDX_SHIFT = 14
IDX_MASK = (1 << IDX_SHIFT) - 1
KA = 64
NBUF = 4
CPA = E_PAD // NW // KA


def _agg_body(h_hbm, pidx_hbm, zeros_hbm, out_hbm,
              idx_v, sstg, dstg, rows0, rows1, rows2, rows3,
              g0, g1, g2, g3, acc_sh):
    rows = (rows0, rows1, rows2, rows3)
    gsem = (g0, g1, g2, g3)
    c = lax.axis_index("c")
    s = lax.axis_index("s")
    w = c * NS + s
    sl = pl.ds(s * RPS, RPS)

    pltpu.sync_copy(zeros_hbm.at[sl], acc_sh.at[sl])

    pltpu.sync_copy(pidx_hbm.at[w], idx_v)
    plsc.subcore_barrier()

    def decode(j, p):
        for i in range(KA // 16):
            ds = pl.ds(i * 16, 16)
            pk = idx_v[j, ds]
            sstg[p, ds] = jnp.right_shift(pk, IDX_SHIFT)
            dstg[p, ds] = jnp.bitwise_and(pk, IDX_MASK)

    def gather_start(b):
        pltpu.async_copy(h_hbm.at[sstg.at[b]], rows[b], gsem[b])

    def gather_wait(b):
        pltpu.make_async_copy(h_hbm.at[sstg.at[b]], rows[b], gsem[b]).wait()

    def scatter(b):
        pltpu.sync_copy(rows[b], acc_sh.at[dstg.at[b]], add=True)

    for b in range(NBUF):
        decode(b, b)
        gather_start(b)

    def group_body(g, carry):
        for b in range(NBUF):
            j = g * NBUF + b
            gather_wait(b)
            scatter(b)
            decode(j + NBUF, b)
            gather_start(b)
        return carry

    lax.fori_loop(0, CPA // NBUF - 1, group_body, 0)
    for b in range(NBUF):
        gather_wait(b)
        scatter(b)
    plsc.subcore_barrier()
    pltpu.sync_copy(acc_sh.at[sl], out_hbm.at[c].at[sl])


def _make_agg(d):
    return pl.kernel(
        _agg_body,
        out_type=jax.ShapeDtypeStruct((NC, N_PAD, d), jnp.float32),
        mesh=_mesh,
        compiler_params=pltpu.CompilerParams(use_tc_tiling_on_sc=False),
        scratch_types=[
            pltpu.VMEM((CPA, KA), jnp.int32),
            pltpu.VMEM((NBUF, KA), jnp.int32),
            pltpu.VMEM((NBUF, KA), jnp.int32),
        ] + [pltpu.VMEM((KA, d), jnp.float32) for _ in range(NBUF)]
          + [pltpu.SemaphoreType.DMA for _ in range(NBUF)]
          + [pltpu.VMEM_SHARED((N_PAD, d), jnp.float32)],
    )


_agg128 = _make_agg(D_H)
_agg64 = _make_agg(D_OUT)


BLK = 512
GRID = N_PAD // BLK


def _deg_mm_body(c0_ref, c1_ref, x_ref, w_ref, h_ref, dinv_ref):
    cnt = c0_ref[:, 0:1] + c1_ref[:, 0:1] + 1.0
    d = lax.rsqrt(cnt)
    h = jnp.dot(x_ref[...], w_ref[...], preferred_element_type=jnp.float32)
    h_ref[...] = h * d
    dinv_ref[...] = jnp.broadcast_to(d, (BLK, CW))


_deg_mm = pl.pallas_call(
    _deg_mm_body,
    grid=(GRID,),
    in_specs=[
        pl.BlockSpec((BLK, CW), lambda i: (i, 0)),
        pl.BlockSpec((BLK, CW), lambda i: (i, 0)),
        pl.BlockSpec((BLK, D_IN), lambda i: (i, 0)),
        pl.BlockSpec((D_IN, D_H), lambda i: (0, 0)),
    ],
    out_specs=[
        pl.BlockSpec((BLK, D_H), lambda i: (i, 0)),
        pl.BlockSpec((BLK, CW), lambda i: (i, 0)),
    ],
    out_shape=[
        jax.ShapeDtypeStruct((N_PAD, D_H), jnp.float32),
        jax.ShapeDtypeStruct((N_PAD, CW), jnp.float32),
    ],
)


def _mid_body(p0_ref, p1_ref, hp_ref, dinv_ref, b_ref, w_ref, out_ref):
    d = dinv_ref[:, 0:1]
    r = jnp.maximum((p0_ref[...] + p1_ref[...] + hp_ref[...]) * d + b_ref[...], 0.0)
    out_ref[...] = jnp.dot(r, w_ref[...], preferred_element_type=jnp.float32) * d


_mid = pl.pallas_call(
    _mid_body,
    grid=(GRID,),
    in_specs=[
        pl.BlockSpec((BLK, D_H), lambda i: (i, 0)),
        pl.BlockSpec((BLK, D_H), lambda i: (i, 0)),
        pl.BlockSpec((BLK, D_H), lambda i: (i, 0)),
        pl.BlockSpec((BLK, CW), lambda i: (i, 0)),
        pl.BlockSpec((1, D_H), lambda i: (0, 0)),
        pl.BlockSpec((D_H, D_OUT), lambda i: (0, 0)),
    ],
    out_specs=pl.BlockSpec((BLK, D_OUT), lambda i: (i, 0)),
    out_shape=jax.ShapeDtypeStruct((N_PAD, D_OUT), jnp.float32),
)


def _final_body(q0_ref, q1_ref, hp_ref, dinv_ref, b_ref, out_ref):
    d = dinv_ref[:, 0:1]
    out_ref[...] = (q0_ref[...] + q1_ref[...] + hp_ref[...]) * d + b_ref[...]


_final = pl.pallas_call(
    _final_body,
    grid=(GRID,),
    in_specs=[
        pl.BlockSpec((BLK, D_OUT), lambda i: (i, 0)),
        pl.BlockSpec((BLK, D_OUT), lambda i: (i, 0)),
        pl.BlockSpec((BLK, D_OUT), lambda i: (i, 0)),
        pl.BlockSpec((BLK, CW), lambda i: (i, 0)),
        pl.BlockSpec((1, D_OUT), lambda i: (0, 0)),
    ],
    out_specs=pl.BlockSpec((BLK, D_OUT), lambda i: (i, 0)),
    out_shape=jax.ShapeDtypeStruct((N_PAD, D_OUT), jnp.float32),
)


def kernel(x, edge_index, train_mask, labels, W1, b1, W2, b2):
    del train_mask, labels
    padv = jnp.full((E_PAD - edge_index.shape[1],), N_PAD - 1, dtype=jnp.int32)
    src = jnp.concatenate([edge_index[0], padv]).reshape(NW, CPW, K)
    dst = jnp.concatenate([edge_index[1], padv]).reshape(NW, CPW, K)
    pidx = jnp.bitwise_or(jnp.left_shift(src, IDX_SHIFT), dst).reshape(NW, CPA, KA)

    x_pad = jnp.pad(x, ((0, N_PAD - N), (0, 0)))
    zeros_cnt = jnp.zeros((N_PAD, CW), jnp.float32)
    ones_k = jnp.ones((K, CW), jnp.float32)
    zeros_h = jnp.zeros((N_PAD, D_H), jnp.float32)
    zeros_o = jnp.zeros((N_PAD, D_OUT), jnp.float32)

    counts = _hist(dst, zeros_cnt, ones_k)
    h1p, dinv = _deg_mm(counts[0], counts[1], x_pad, W1)
    p = _agg128(h1p, pidx, zeros_h)
    h2p = _mid(p[0], p[1], h1p, dinv, b1.reshape(1, D_H), W2)
    q = _agg64(h2p, pidx, zeros_o)
    out = _final(q[0], q[1], h2p, dinv, b2.reshape(1, D_OUT))
    return out[:N]


# speedup vs baseline: 2.3173x; 2.0398x over previous
import functools

import jax
import jax.numpy as jnp
from jax import lax
from jax.experimental import pallas as pl
from jax.experimental.pallas import tpu as pltpu
from jax.experimental.pallas import tpu_sc as plsc

N = 10000
D_IN = 128
D_H = 128
D_OUT = 64

NC = 2
NS = 16
NW = NC * NS
K = 128
CPW = 80
E_PAD = NW * CPW * K
N_PAD = 10240
RPS = N_PAD // NS
CW = 16

_mesh = plsc.VectorSubcoreMesh(core_axis_name="c", subcore_axis_name="s")


def _hist_body(dst_hbm, zeros_hbm, ones_hbm, out_hbm, dst_v, ones_v, cnt_sh):
    c = lax.axis_index("c")
    s = lax.axis_index("s")
    w = c * NS + s
    sl = pl.ds(s * RPS, RPS)
    pltpu.sync_copy(zeros_hbm.at[sl], cnt_sh.at[sl])
    pltpu.sync_copy(ones_hbm, ones_v)
    pltpu.sync_copy(dst_hbm.at[w], dst_v)
    plsc.subcore_barrier()

    def body(j, carry):
        pltpu.sync_copy(ones_v, cnt_sh.at[dst_v.at[j]], add=True)
        return carry

    lax.fori_loop(0, CPW, body, 0)
    plsc.subcore_barrier()
    pltpu.sync_copy(cnt_sh.at[sl], out_hbm.at[c].at[sl])


_hist = pl.kernel(
    _hist_body,
    out_type=jax.ShapeDtypeStruct((NC, N_PAD, CW), jnp.float32),
    mesh=_mesh,
    compiler_params=pltpu.CompilerParams(use_tc_tiling_on_sc=False),
    scratch_types=[
        pltpu.VMEM((CPW, K), jnp.int32),
        pltpu.VMEM((K, CW), jnp.float32),
        pltpu.VMEM_SHARED((N_PAD, CW), jnp.float32),
    ],
)


IDX_SHIFT = 14
IDX_MASK = (1 << IDX_SHIFT) - 1
KA = 64
NBUF = 4
CPT = E_PAD // NS // KA


def _agg_body(hsplit_hbm, pidx_hbm, out_hbm,
              idx_v, sstg, dstg, rows0, rows1, rows2, rows3,
              g0, g1, g2, g3, table_sh, acc_sh):
    rows = (rows0, rows1, rows2, rows3)
    gsem = (g0, g1, g2, g3)
    c = lax.axis_index("c")
    s = lax.axis_index("s")
    sl = pl.ds(s * RPS, RPS)

    pltpu.sync_copy(hsplit_hbm.at[c].at[sl], table_sh.at[sl])
    pltpu.sync_copy(hsplit_hbm.at[c].at[sl], acc_sh.at[sl])
    pltpu.sync_copy(pidx_hbm.at[s], idx_v)
    plsc.subcore_barrier()

    def decode(j, p):
        for i in range(KA // 16):
            ds = pl.ds(i * 16, 16)
            pk = idx_v[j, ds]
            sstg[p, ds] = jnp.right_shift(pk, IDX_SHIFT)
            dstg[p, ds] = jnp.bitwise_and(pk, IDX_MASK)

    def gather_start(b):
        pltpu.async_copy(table_sh.at[sstg.at[b]], rows[b], gsem[b])

    def gather_wait(b):
        pltpu.make_async_copy(table_sh.at[sstg.at[b]], rows[b], gsem[b]).wait()

    def scatter(b):
        pltpu.sync_copy(rows[b], acc_sh.at[dstg.at[b]], add=True)

    for b in range(NBUF):
        decode(b, b)
        gather_start(b)

    def group_body(g, carry):
        for b in range(NBUF):
            j = g * NBUF + b
            gather_wait(b)
            scatter(b)
            decode(j + NBUF, b)
            gather_start(b)
        return carry

    lax.fori_loop(0, CPT // NBUF - 1, group_body, 0)
    for b in range(NBUF):
        gather_wait(b)
        scatter(b)
    plsc.subcore_barrier()
    pltpu.sync_copy(acc_sh.at[sl], out_hbm.at[c].at[sl])


def _make_agg(dh):
    return pl.kernel(
        _agg_body,
        out_type=jax.ShapeDtypeStruct((NC, N_PAD, dh), jnp.float32),
        mesh=_mesh,
        compiler_params=pltpu.CompilerParams(use_tc_tiling_on_sc=False),
        scratch_types=[
            pltpu.VMEM((CPT, KA), jnp.int32),
            pltpu.VMEM((NBUF, KA), jnp.int32),
            pltpu.VMEM((NBUF, KA), jnp.int32),
        ] + [pltpu.VMEM((KA, dh), jnp.float32) for _ in range(NBUF)]
          + [pltpu.SemaphoreType.DMA for _ in range(NBUF)]
          + [pltpu.VMEM_SHARED((N_PAD, dh), jnp.float32),
             pltpu.VMEM_SHARED((N_PAD, dh), jnp.float32)],
    )


_agg128 = _make_agg(D_H // 2)
_agg64 = _make_agg(D_OUT // 2)


BLK = 512
GRID = N_PAD // BLK


def _deg_mm_body(c0_ref, c1_ref, x_ref, w_ref, h_ref, dinv_ref):
    cnt = c0_ref[:, 0:1] + c1_ref[:, 0:1] + 1.0
    d = lax.rsqrt(cnt)
    h = jnp.dot(x_ref[...], w_ref[...], preferred_element_type=jnp.float32) * d
    h_ref[0] = h[:, :D_H // 2]
    h_ref[1] = h[:, D_H // 2:]
    dinv_ref[...] = jnp.broadcast_to(d, (BLK, CW))


_deg_mm = pl.pallas_call(
    _deg_mm_body,
    grid=(GRID,),
    in_specs=[
        pl.BlockSpec((BLK, CW), lambda i: (i, 0)),
        pl.BlockSpec((BLK, CW), lambda i: (i, 0)),
        pl.BlockSpec((BLK, D_IN), lambda i: (i, 0)),
        pl.BlockSpec((D_IN, D_H), lambda i: (0, 0)),
    ],
    out_specs=[
        pl.BlockSpec((NC, BLK, D_H // 2), lambda i: (0, i, 0)),
        pl.BlockSpec((BLK, CW), lambda i: (i, 0)),
    ],
    out_shape=[
        jax.ShapeDtypeStruct((NC, N_PAD, D_H // 2), jnp.float32),
        jax.ShapeDtypeStruct((N_PAD, CW), jnp.float32),
    ],
)


def _mid_body(p_ref, dinv_ref, b_ref, w_ref, h2_ref):
    d = dinv_ref[:, 0:1]
    u = jnp.concatenate([p_ref[0], p_ref[1]], axis=1)
    r = jnp.maximum(u * d + b_ref[...], 0.0)
    h2 = jnp.dot(r, w_ref[...], preferred_element_type=jnp.float32) * d
    h2_ref[0] = h2[:, :D_OUT // 2]
    h2_ref[1] = h2[:, D_OUT // 2:]


_mid = pl.pallas_call(
    _mid_body,
    grid=(GRID,),
    in_specs=[
        pl.BlockSpec((NC, BLK, D_H // 2), lambda i: (0, i, 0)),
        pl.BlockSpec((BLK, CW), lambda i: (i, 0)),
        pl.BlockSpec((1, D_H), lambda i: (0, 0)),
        pl.BlockSpec((D_H, D_OUT), lambda i: (0, 0)),
    ],
    out_specs=pl.BlockSpec((NC, BLK, D_OUT // 2), lambda i: (0, i, 0)),
    out_shape=jax.ShapeDtypeStruct((NC, N_PAD, D_OUT // 2), jnp.float32),
)


def _final_body(q_ref, dinv_ref, b_ref, out_ref):
    d = dinv_ref[:, 0:1]
    u = jnp.concatenate([q_ref[0], q_ref[1]], axis=1)
    out_ref[...] = u * d + b_ref[...]


_final = pl.pallas_call(
    _final_body,
    grid=(GRID,),
    in_specs=[
        pl.BlockSpec((NC, BLK, D_OUT // 2), lambda i: (0, i, 0)),
        pl.BlockSpec((BLK, CW), lambda i: (i, 0)),
        pl.BlockSpec((1, D_OUT), lambda i: (0, 0)),
    ],
    out_specs=pl.BlockSpec((BLK, D_OUT), lambda i: (i, 0)),
    out_shape=jax.ShapeDtypeStruct((N_PAD, D_OUT), jnp.float32),
)


def kernel(x, edge_index, train_mask, labels, W1, b1, W2, b2):
    del train_mask, labels
    padv = jnp.full((E_PAD - edge_index.shape[1],), N_PAD - 1, dtype=jnp.int32)
    src = jnp.concatenate([edge_index[0], padv])
    dst = jnp.concatenate([edge_index[1], padv])
    pidx = jnp.bitwise_or(jnp.left_shift(src, IDX_SHIFT), dst).reshape(NS, CPT, KA)
    dst_h = dst.reshape(NW, E_PAD // NW // K, K)

    x_pad = jnp.pad(x, ((0, N_PAD - N), (0, 0)))
    zeros_cnt = jnp.zeros((N_PAD, CW), jnp.float32)
    ones_k = jnp.ones((K, CW), jnp.float32)

    counts = _hist(dst_h, zeros_cnt, ones_k)
    hsplit, dinv = _deg_mm(counts[0], counts[1], x_pad, W1)
    p = _agg128(hsplit, pidx)
    h2split = _mid(p, dinv, b1.reshape(1, D_H), W2)
    q = _agg64(h2split, pidx)
    out = _final(q, dinv, b2.reshape(1, D_OUT))
    return out[:N]


# speedup vs baseline: 2.3465x; 1.0126x over previous
import functools

import jax
import jax.numpy as jnp
from jax import lax
from jax.experimental import pallas as pl
from jax.experimental.pallas import tpu as pltpu
from jax.experimental.pallas import tpu_sc as plsc

N = 10000
D_IN = 128
D_H = 128
D_OUT = 64

NC = 2
NS = 16
NW = NC * NS
K = 128
CPW = 80
E_PAD = NW * CPW * K
N_PAD = 10240
RPS = N_PAD // NS
CW = 16

_mesh = plsc.VectorSubcoreMesh(core_axis_name="c", subcore_axis_name="s")


def _hist_body(dst_hbm, zeros_hbm, ones_hbm, out_hbm, dst_v, ones_v, cnt_sh):
    c = lax.axis_index("c")
    s = lax.axis_index("s")
    w = c * NS + s
    sl = pl.ds(s * RPS, RPS)
    pltpu.sync_copy(zeros_hbm.at[sl], cnt_sh.at[sl])
    pltpu.sync_copy(ones_hbm, ones_v)
    pltpu.sync_copy(dst_hbm.at[w], dst_v)
    plsc.subcore_barrier()

    def body(j, carry):
        pltpu.sync_copy(ones_v, cnt_sh.at[dst_v.at[j]], add=True)
        return carry

    lax.fori_loop(0, CPW, body, 0)
    plsc.subcore_barrier()
    pltpu.sync_copy(cnt_sh.at[sl], out_hbm.at[c].at[sl])


_hist = pl.kernel(
    _hist_body,
    out_type=jax.ShapeDtypeStruct((NC, N_PAD, CW), jnp.float32),
    mesh=_mesh,
    compiler_params=pltpu.CompilerParams(use_tc_tiling_on_sc=False),
    scratch_types=[
        pltpu.VMEM((CPW, K), jnp.int32),
        pltpu.VMEM((K, CW), jnp.float32),
        pltpu.VMEM_SHARED((N_PAD, CW), jnp.float32),
    ],
)


IDX_SHIFT = 14
IDX_MASK = (1 << IDX_SHIFT) - 1
KA = 64
NBUF = 4
CPT = E_PAD // NS // KA


def _agg_body(hsplit_hbm, pidx_hbm, out_hbm,
              idx_v, sstg, dstg, rows0, rows1, rows2, rows3,
              g0, g1, g2, g3, table_sh, acc_sh):
    rows = (rows0, rows1, rows2, rows3)
    gsem = (g0, g1, g2, g3)
    c = lax.axis_index("c")
    s = lax.axis_index("s")
    sl = pl.ds(s * RPS, RPS)

    pltpu.sync_copy(hsplit_hbm.at[c].at[sl], table_sh.at[sl])
    pltpu.sync_copy(hsplit_hbm.at[c].at[sl], acc_sh.at[sl])
    pltpu.sync_copy(pidx_hbm.at[s], idx_v)
    plsc.subcore_barrier()

    def decode(j, p):
        for i in range(KA // 16):
            ds = pl.ds(i * 16, 16)
            pk = idx_v[j, ds]
            sstg[p, ds] = jnp.right_shift(pk, IDX_SHIFT)
            dstg[p, ds] = jnp.bitwise_and(pk, IDX_MASK)

    def gather_start(b):
        pltpu.async_copy(table_sh.at[sstg.at[b]], rows[b], gsem[b])

    def gather_wait(b):
        pltpu.make_async_copy(table_sh.at[sstg.at[b]], rows[b], gsem[b]).wait()

    def scatter(b):
        pltpu.sync_copy(rows[b], acc_sh.at[dstg.at[b]], add=True)

    for b in range(NBUF):
        decode(b, b)
        gather_start(b)

    def group_body(g, carry):
        for b in range(NBUF):
            j = g * NBUF + b
            gather_wait(b)
            scatter(b)
            decode(j + NBUF, b)
            gather_start(b)
        return carry

    lax.fori_loop(0, CPT // NBUF - 1, group_body, 0)
    for b in range(NBUF):
        gather_wait(b)
        scatter(b)
    plsc.subcore_barrier()
    pltpu.sync_copy(acc_sh.at[sl], out_hbm.at[c].at[sl])


def _make_agg(dh):
    return pl.kernel(
        _agg_body,
        out_type=jax.ShapeDtypeStruct((NC, N_PAD, dh), jnp.float32),
        mesh=_mesh,
        compiler_params=pltpu.CompilerParams(use_tc_tiling_on_sc=False),
        scratch_types=[
            pltpu.VMEM((CPT, KA), jnp.int32),
            pltpu.VMEM((NBUF, KA), jnp.int32),
            pltpu.VMEM((NBUF, KA), jnp.int32),
        ] + [pltpu.VMEM((KA, dh), jnp.float32) for _ in range(NBUF)]
          + [pltpu.SemaphoreType.DMA for _ in range(NBUF)]
          + [pltpu.VMEM_SHARED((N_PAD, dh), jnp.float32),
             pltpu.VMEM_SHARED((N_PAD, dh), jnp.float32)],
    )


_agg128 = _make_agg(D_H // 2)
_agg64 = _make_agg(D_OUT // 2)


BLK = 512
GRID = N_PAD // BLK


def _deg_mm_body(cnt_ref, x_ref, w_ref, h_ref, dinv_ref):
    cnt = cnt_ref[0][:, 0:1] + cnt_ref[1][:, 0:1] + 1.0
    d = lax.rsqrt(cnt)
    h = jnp.dot(x_ref[...], w_ref[...], preferred_element_type=jnp.float32) * d
    h_ref[0] = h[:, :D_H // 2]
    h_ref[1] = h[:, D_H // 2:]
    dinv_ref[...] = jnp.broadcast_to(d, (BLK, CW))


_deg_mm = pl.pallas_call(
    _deg_mm_body,
    grid=(GRID,),
    in_specs=[
        pl.BlockSpec((NC, BLK, CW), lambda i: (0, i, 0)),
        pl.BlockSpec((BLK, D_IN), lambda i: (i, 0)),
        pl.BlockSpec((D_IN, D_H), lambda i: (0, 0)),
    ],
    out_specs=[
        pl.BlockSpec((NC, BLK, D_H // 2), lambda i: (0, i, 0)),
        pl.BlockSpec((BLK, CW), lambda i: (i, 0)),
    ],
    out_shape=[
        jax.ShapeDtypeStruct((NC, N_PAD, D_H // 2), jnp.float32),
        jax.ShapeDtypeStruct((N_PAD, CW), jnp.float32),
    ],
)


def _mid_body(p_ref, dinv_ref, b_ref, w_ref, h2_ref):
    d = dinv_ref[:, 0:1]
    u = jnp.concatenate([p_ref[0], p_ref[1]], axis=1)
    r = jnp.maximum(u * d + b_ref[...], 0.0)
    h2 = jnp.dot(r, w_ref[...], preferred_element_type=jnp.float32) * d
    h2_ref[0] = h2[:, :D_OUT // 2]
    h2_ref[1] = h2[:, D_OUT // 2:]


_mid = pl.pallas_call(
    _mid_body,
    grid=(GRID,),
    in_specs=[
        pl.BlockSpec((NC, BLK, D_H // 2), lambda i: (0, i, 0)),
        pl.BlockSpec((BLK, CW), lambda i: (i, 0)),
        pl.BlockSpec((1, D_H), lambda i: (0, 0)),
        pl.BlockSpec((D_H, D_OUT), lambda i: (0, 0)),
    ],
    out_specs=pl.BlockSpec((NC, BLK, D_OUT // 2), lambda i: (0, i, 0)),
    out_shape=jax.ShapeDtypeStruct((NC, N_PAD, D_OUT // 2), jnp.float32),
)


def _final_body(q_ref, dinv_ref, b_ref, out_ref):
    d = dinv_ref[:, 0:1]
    u = jnp.concatenate([q_ref[0], q_ref[1]], axis=1)
    out_ref[...] = u * d + b_ref[...]


BLKF = 400

_final = pl.pallas_call(
    _final_body,
    grid=(N // BLKF,),
    in_specs=[
        pl.BlockSpec((NC, BLKF, D_OUT // 2), lambda i: (0, i, 0)),
        pl.BlockSpec((BLKF, CW), lambda i: (i, 0)),
        pl.BlockSpec((1, D_OUT), lambda i: (0, 0)),
    ],
    out_specs=pl.BlockSpec((BLKF, D_OUT), lambda i: (i, 0)),
    out_shape=jax.ShapeDtypeStruct((N, D_OUT), jnp.float32),
)


def kernel(x, edge_index, train_mask, labels, W1, b1, W2, b2):
    del train_mask, labels
    padv = jnp.full((E_PAD - edge_index.shape[1],), N_PAD - 1, dtype=jnp.int32)
    src = jnp.concatenate([edge_index[0], padv])
    dst = jnp.concatenate([edge_index[1], padv])
    pidx = jnp.bitwise_or(jnp.left_shift(src, IDX_SHIFT), dst).reshape(NS, CPT, KA)
    dst_h = dst.reshape(NW, E_PAD // NW // K, K)

    x_pad = jnp.pad(x, ((0, N_PAD - N), (0, 0)))
    zeros_cnt = jnp.zeros((N_PAD, CW), jnp.float32)
    ones_k = jnp.ones((K, CW), jnp.float32)

    counts = _hist(dst_h, zeros_cnt, ones_k)
    hsplit, dinv = _deg_mm(counts, x_pad, W1)
    p = _agg128(hsplit, pidx)
    h2split = _mid(p, dinv, b1.reshape(1, D_H), W2)
    q = _agg64(h2split, pidx)
    return _final(q, dinv, b2.reshape(1, D_OUT))
